# Initial kernel scaffold; baseline (speedup 1.0000x reference)
import jax
import jax.numpy as jnp
from jax.experimental import pallas as pl


def kernel(tokens, w_e):
    raise NotImplementedError("write your pallas kernel here")


# speedup vs baseline: 1.8434x; 1.8434x over previous
import functools

import jax
import jax.numpy as jnp
from jax import lax
from jax.experimental import pallas as pl
from jax.experimental.pallas import tpu as pltpu
from jax.experimental.pallas import tpu_sc as plsc

NC = 2
NS = 16
NW = NC * NS

D = 64
CH = 1024
G = 128
IB = CH // G


@functools.cache
def _build(B, V):
    b_per_w = B // NW
    n_chunks = b_per_w // CH
    mesh = plsc.VectorSubcoreMesh(
        core_axis_name="c", subcore_axis_name="s",
        num_cores=NC, num_subcores=NS)

    @functools.partial(
        pl.kernel,
        out_type=jax.ShapeDtypeStruct((B, D), jnp.float32),
        mesh=mesh,
        scratch_types=[
            pltpu.VMEM((IB, G), jnp.int32),
            pltpu.VMEM((CH, D), jnp.float32),
            pltpu.SemaphoreType.DMA,
        ],
        compiler_params=pltpu.CompilerParams(use_tc_tiling_on_sc=False),
    )
    def k(idx_hbm, table_hbm, out_hbm, idx_v, rows_v, sem):
        wid = lax.axis_index("s") * NC + lax.axis_index("c")
        base = wid * b_per_w

        def body(c, _):
            r0 = pl.multiple_of(base + c * CH, CH)
            pltpu.sync_copy(idx_hbm.at[pl.ds(pl.multiple_of(r0 // G, IB), IB)], idx_v)
            copies = [
                pltpu.async_copy(
                    table_hbm.at[idx_v.at[j]],
                    rows_v.at[pl.ds(j * G, G)],
                    sem,
                )
                for j in range(IB)
            ]
            for cp in copies:
                cp.wait()
            pltpu.sync_copy(rows_v, out_hbm.at[pl.ds(r0, CH)])
            return ()

        lax.fori_loop(0, n_chunks, body, ())

    return k


def kernel(tokens, w_e):
    n, s = tokens.shape
    B = n * s
    idx = tokens.reshape(B // G, G).astype(jnp.int32)
    out = _build(B, w_e.shape[0])(idx, w_e)
    return out.reshape(n, s, D)


# speedup vs baseline: 1.8742x; 1.0167x over previous
import functools

import jax
import jax.numpy as jnp
from jax import lax
from jax.experimental import pallas as pl
from jax.experimental.pallas import tpu as pltpu
from jax.experimental.pallas import tpu_sc as plsc

NC = 2
NS = 16
NW = NC * NS

D = 64
CH = 640
G = 128
IB = CH // G


@functools.cache
def _build(B, V):
    b_per_w = B // NW
    n_chunks = b_per_w // CH
    assert n_chunks % 2 == 0
    mesh = plsc.VectorSubcoreMesh(
        core_axis_name="c", subcore_axis_name="s",
        num_cores=NC, num_subcores=NS)

    @functools.partial(
        pl.kernel,
        out_type=jax.ShapeDtypeStruct((B, D), jnp.float32),
        mesh=mesh,
        scratch_types=[
            pltpu.VMEM((b_per_w,), jnp.int32),
            pltpu.VMEM((CH, D), jnp.float32),
            pltpu.VMEM((CH, D), jnp.float32),
            pltpu.SemaphoreType.DMA,
            pltpu.SemaphoreType.DMA,
            pltpu.SemaphoreType.DMA,
        ],
        compiler_params=pltpu.CompilerParams(use_tc_tiling_on_sc=False),
    )
    def k(idx_hbm, table_hbm, out_hbm, idx_v, rows0, rows1, gsem, osem0, osem1):
        wid = lax.axis_index("s") * NC + lax.axis_index("c")
        base = pl.multiple_of(wid * b_per_w, b_per_w)
        rows = (rows0, rows1)
        osem = (osem0, osem1)

        pltpu.sync_copy(idx_hbm.at[pl.ds(base, b_per_w)], idx_v)

        def gather_copies(t, buf):
            off = pl.multiple_of(t * CH, CH)
            return [
                pltpu.make_async_copy(
                    table_hbm.at[idx_v.at[pl.ds(off + j * G, G)]],
                    buf.at[pl.ds(j * G, G)],
                    gsem,
                )
                for j in range(IB)
            ]

        def out_copy(t, b):
            off = pl.multiple_of(base + t * CH, CH)
            return pltpu.make_async_copy(rows[b], out_hbm.at[pl.ds(off, CH)],
                                         osem[b])

        for cp in gather_copies(0, rows[0]):
            cp.start()

        @pl.loop(0, n_chunks, step=2)
        def _(c):
            for b in (0, 1):
                t = c + b
                for cp in gather_copies(t, rows[b]):
                    cp.wait()
                out_copy(t, b).start()

                @pl.when(t >= 1)
                def _():
                    out_copy(t - 1, 1 - b).wait()

                @pl.when(t + 1 < n_chunks)
                def _():
                    for cp in gather_copies(t + 1, rows[1 - b]):
                        cp.start()

        out_copy(n_chunks - 1, 1).wait()

    return k


def kernel(tokens, w_e):
    n, s = tokens.shape
    B = n * s
    idx = tokens.reshape(B).astype(jnp.int32)
    out = _build(B, w_e.shape[0])(idx, w_e)
    return out.reshape(n, s, D)
